# Initial kernel scaffold; baseline (speedup 1.0000x reference)
import jax
import jax.numpy as jnp
from jax.experimental import pallas as pl


def kernel(s, v, pos, node_type, segment_ids, atomic_mass, Wv1_0, Wv2_0, W1_0, b1_0, W2_0, b2_0, Wv1_1, Wv2_1, W1_1, b1_1, W2_1, b2_1):
    raise NotImplementedError("write your pallas kernel here")


# speedup vs baseline: 1.4336x; 1.4336x over previous
import jax
import jax.numpy as jnp
from jax.experimental import pallas as pl
from jax.experimental.pallas import tpu as pltpu


def _body(s_ref, v_ref, pos_ref, nt_ref, seg_ref, am_ref,
          Wv1_0_ref, Wv2_0_ref, Ws_ref, Wn_ref, b1_0_ref, W2_0_ref, b2_0_ref,
          Wv1_1_ref, Wv2_1_ref, W11a_ref, W11b_ref, b1_1_ref, W2_1_ref, b2_1_ref,
          out_ref, acc_ref):
    i = pl.program_id(0)
    blk = s_ref.shape[0]
    nseg = acc_ref.shape[1]

    @pl.when(i == 0)
    def _init():
        acc_ref[...] = jnp.zeros_like(acc_ref)

    def dot(a, b):
        return jnp.dot(a, b, preferred_element_type=jnp.float32)

    def dot_hi(a, b):
        return jnp.dot(a, b, preferred_element_type=jnp.float32,
                       precision=jax.lax.Precision.HIGHEST)

    s = s_ref[...]
    vx = v_ref[:, 0:128]
    vy = v_ref[:, 128:256]
    vz = v_ref[:, 256:384]

    Wv1 = Wv1_0_ref[...]
    a1x = dot(vx, Wv1); a1y = dot(vy, Wv1); a1z = dot(vz, Wv1)
    vec1n = jnp.sqrt(a1x * a1x + a1y * a1y + a1z * a1z + 1e-12)
    Wv2 = Wv2_0_ref[...]
    a2x = dot(vx, Wv2); a2y = dot(vy, Wv2); a2z = dot(vz, Wv2)
    h = dot(s, Ws_ref[...]) + dot(vec1n, Wn_ref[...]) + b1_0_ref[...]
    h = h * jax.nn.sigmoid(h)
    h2 = dot(h, W2_0_ref[...]) + b2_0_ref[...]
    s0 = h2[:, 0:64]
    gate = h2[:, 64:128]
    s0 = s0 * jax.nn.sigmoid(s0)
    bx = a2x * gate; by = a2y * gate; bz = a2z * gate

    Wv1b = Wv1_1_ref[...]
    c1x = dot_hi(bx, Wv1b); c1y = dot_hi(by, Wv1b); c1z = dot_hi(bz, Wv1b)
    vec1n2 = jnp.sqrt(c1x * c1x + c1y * c1y + c1z * c1z + 1e-12)
    Wv2b = Wv2_1_ref[...]
    c2x = dot_hi(bx, Wv2b); c2y = dot_hi(by, Wv2b); c2z = dot_hi(bz, Wv2b)
    hh = dot(s0, W11a_ref[...]) + vec1n2 * W11b_ref[...] + b1_1_ref[...]
    hh = hh * jax.nn.sigmoid(hh)
    h3 = dot_hi(hh, W2_1_ref[...]) + b2_1_ref[...]
    s_fin = h3[:, 0:1]
    g2 = h3[:, 1:2]
    vfx = c2x * g2; vfy = c2y * g2; vfz = c2z * g2

    nt = nt_ref[...]
    t_iota = jax.lax.broadcasted_iota(jnp.int32, (blk, 128), 1)
    nt_oh = (nt == t_iota).astype(jnp.float32)
    am = dot_hi(nt_oh, am_ref[...])

    px = pos_ref[:, 0:1]; py = pos_ref[:, 1:2]; pz = pos_ref[:, 2:3]
    payload = jnp.concatenate([
        vfx, vfy, vfz,
        s_fin * px, s_fin * py, s_fin * pz,
        s_fin, am, am * px, am * py, am * pz,
        jnp.zeros((blk, 5), jnp.float32)], axis=1)

    seg = seg_ref[...]
    b_iota = jax.lax.broadcasted_iota(jnp.int32, (blk, nseg), 1)
    oh = (seg == b_iota).astype(jnp.float32)
    acc_ref[...] += jax.lax.dot_general(
        payload, oh, (((0,), (0,)), ((), ())),
        preferred_element_type=jnp.float32,
        precision=jax.lax.Precision.HIGHEST)

    @pl.when(i == pl.num_programs(0) - 1)
    def _epilogue():
        A = acc_ref[...]
        vsum = A[0:3, :]
        spsum = A[3:6, :]
        ssum = A[6:7, :]
        amsum = A[7:8, :]
        msum = A[8:11, :]
        denom = jnp.where(amsum == 0.0, 1.0, amsum)
        mc = msum / denom
        o = vsum + spsum - ssum * mc
        out_ref[...] = jnp.sqrt(jnp.sum(o * o, axis=0, keepdims=True) + 1e-12)


def kernel(s, v, pos, node_type, segment_ids, atomic_mass,
           Wv1_0, Wv2_0, W1_0, b1_0, W2_0, b2_0,
           Wv1_1, Wv2_1, W1_1, b1_1, W2_1, b2_1):
    N, F = s.shape
    B = 1024
    blk = 1000 if N % 1000 == 0 else (N if N <= 1000 else None)
    assert blk is not None and N % blk == 0
    grid = N // blk

    v2 = v.reshape(N, 3 * F)
    nt2 = node_type.astype(jnp.int32).reshape(N, 1)
    seg2 = segment_ids.astype(jnp.int32).reshape(N, 1)
    am_pad = jnp.zeros((128, 1), jnp.float32).at[:atomic_mass.shape[0], 0].set(atomic_mass)
    Ws = W1_0[:F]
    Wn = W1_0[F:]
    W11a = W1_1[:F // 2]
    W11b = W1_1[F // 2:]

    full = lambda shape: pl.BlockSpec(shape, lambda i: (0,) * len(shape))
    row = lambda shape: pl.BlockSpec(shape, lambda i: (i,) + (0,) * (len(shape) - 1))

    out = pl.pallas_call(
        _body,
        grid=(grid,),
        in_specs=[
            row((blk, F)),
            row((blk, 3 * F)),
            row((blk, 3)),
            row((blk, 1)),
            row((blk, 1)),
            full((128, 1)),
            full((F, F // 2)),
            full((F, F // 2)),
            full((F, F)),
            full((F // 2, F)),
            full((1, F)),
            full((F, 2 * (F // 2))),
            full((1, 2 * (F // 2))),
            full((F // 2, 1)),
            full((F // 2, 1)),
            full((F // 2, F // 2)),
            full((1, F // 2)),
            full((1, F // 2)),
            full((F // 2, 2)),
            full((1, 2)),
        ],
        out_specs=pl.BlockSpec((1, B), lambda i: (0, 0)),
        out_shape=jax.ShapeDtypeStruct((1, B), jnp.float32),
        scratch_shapes=[pltpu.VMEM((16, B), jnp.float32)],
    )(s, v2, pos, nt2, seg2, am_pad,
      Wv1_0, Wv2_0, Ws, Wn, b1_0.reshape(1, F), W2_0, b2_0.reshape(1, 2 * (F // 2)),
      Wv1_1, Wv2_1, W11a, W11b, b1_1.reshape(1, F // 2), W2_1, b2_1.reshape(1, 2))
    return out.reshape(B, 1)


# speedup vs baseline: 3.1964x; 2.2297x over previous
import jax
import jax.numpy as jnp
from jax.experimental import pallas as pl
from jax.experimental.pallas import tpu as pltpu


def _body(s_ref, v_ref, pos_ref, nt_ref, seg_ref, am_ref,
          Wv_ref, Ws_ref, Wn_ref, b1_0_ref, W2_0_ref, b2_0_ref,
          Wc_ref, W11a_ref, W11b_ref, b1_1_ref, W2_1_ref, b2_1_ref,
          out_ref, acc_ref):
    i = pl.program_id(0)
    blk = s_ref.shape[0]
    nseg = acc_ref.shape[1]

    @pl.when(i == 0)
    def _init():
        acc_ref[...] = jnp.zeros_like(acc_ref)

    def dot(a, b):
        return jnp.dot(a, b, preferred_element_type=jnp.float32)

    s = s_ref[...]
    vx = v_ref[:, 0:128]
    vy = v_ref[:, 128:256]
    vz = v_ref[:, 256:384]

    Wv = Wv_ref[...]
    ax = dot(vx, Wv); ay = dot(vy, Wv); az = dot(vz, Wv)
    a1x = ax[:, 0:64]; a2x = ax[:, 64:128]
    a1y = ay[:, 0:64]; a2y = ay[:, 64:128]
    a1z = az[:, 0:64]; a2z = az[:, 64:128]
    vec1n = jnp.sqrt(a1x * a1x + a1y * a1y + a1z * a1z + 1e-12)
    h = dot(s, Ws_ref[...]) + dot(vec1n, Wn_ref[...]) + b1_0_ref[...]
    h = h * jax.nn.sigmoid(h)
    h2 = dot(h, W2_0_ref[...]) + b2_0_ref[...]
    s0 = h2[:, 0:64]
    gate = h2[:, 64:128]
    s0 = s0 * jax.nn.sigmoid(s0)
    bx = a2x * gate; by = a2y * gate; bz = a2z * gate

    Wc = Wc_ref[...]
    cx = dot(bx, Wc); cy = dot(by, Wc); cz = dot(bz, Wc)
    c1x = cx[:, 0:1]; c2x = cx[:, 1:2]
    c1y = cy[:, 0:1]; c2y = cy[:, 1:2]
    c1z = cz[:, 0:1]; c2z = cz[:, 1:2]
    vec1n2 = jnp.sqrt(c1x * c1x + c1y * c1y + c1z * c1z + 1e-12)
    hh = dot(s0, W11a_ref[...]) + vec1n2 * W11b_ref[...] + b1_1_ref[...]
    hh = hh * jax.nn.sigmoid(hh)
    h3 = dot(hh, W2_1_ref[...]) + b2_1_ref[...]
    s_fin = h3[:, 0:1]
    g2 = h3[:, 1:2]
    vfx = c2x * g2; vfy = c2y * g2; vfz = c2z * g2

    nt = nt_ref[...]
    t_iota = jax.lax.broadcasted_iota(jnp.int32, (blk, 128), 1)
    nt_oh = (nt == t_iota).astype(jnp.float32)
    am = dot(nt_oh, am_ref[...])

    px = pos_ref[:, 0:1]; py = pos_ref[:, 1:2]; pz = pos_ref[:, 2:3]
    payload = jnp.concatenate([
        vfx, vfy, vfz,
        s_fin * px, s_fin * py, s_fin * pz,
        s_fin, am, am * px, am * py, am * pz,
        jnp.zeros((blk, 5), jnp.float32)], axis=1)

    seg = seg_ref[...]
    b_iota = jax.lax.broadcasted_iota(jnp.int32, (blk, nseg), 1)
    oh = (seg == b_iota).astype(jnp.float32)
    acc_ref[...] += jax.lax.dot_general(
        payload, oh, (((0,), (0,)), ((), ())),
        preferred_element_type=jnp.float32)

    @pl.when(i == pl.num_programs(0) - 1)
    def _epilogue():
        A = acc_ref[...]
        vsum = A[0:3, :]
        spsum = A[3:6, :]
        ssum = A[6:7, :]
        amsum = A[7:8, :]
        msum = A[8:11, :]
        denom = jnp.where(amsum == 0.0, 1.0, amsum)
        mc = msum / denom
        o = vsum + spsum - ssum * mc
        out_ref[...] = jnp.sqrt(jnp.sum(o * o, axis=0, keepdims=True) + 1e-12)


def kernel(s, v, pos, node_type, segment_ids, atomic_mass,
           Wv1_0, Wv2_0, W1_0, b1_0, W2_0, b2_0,
           Wv1_1, Wv2_1, W1_1, b1_1, W2_1, b2_1):
    N, F = s.shape
    B = 1024
    blk = 1000 if N % 1000 == 0 else (N if N <= 1000 else None)
    assert blk is not None and N % blk == 0
    grid = N // blk

    v2 = v.reshape(N, 3 * F)
    nt2 = node_type.astype(jnp.int32).reshape(N, 1)
    seg2 = segment_ids.astype(jnp.int32).reshape(N, 1)
    am_pad = jnp.zeros((128, 1), jnp.float32).at[:atomic_mass.shape[0], 0].set(atomic_mass)
    Wv = jnp.concatenate([Wv1_0, Wv2_0], axis=1)
    Wc = jnp.concatenate([Wv1_1, Wv2_1], axis=1)
    Ws = W1_0[:F]
    Wn = W1_0[F:]
    W11a = W1_1[:F // 2]
    W11b = W1_1[F // 2:]

    full = lambda shape: pl.BlockSpec(shape, lambda i: (0,) * len(shape))
    row = lambda shape: pl.BlockSpec(shape, lambda i: (i,) + (0,) * (len(shape) - 1))

    out = pl.pallas_call(
        _body,
        grid=(grid,),
        in_specs=[
            row((blk, F)),
            row((blk, 3 * F)),
            row((blk, 3)),
            row((blk, 1)),
            row((blk, 1)),
            full((128, 1)),
            full((F, F)),
            full((F, F)),
            full((F // 2, F)),
            full((1, F)),
            full((F, 2 * (F // 2))),
            full((1, 2 * (F // 2))),
            full((F // 2, 2)),
            full((F // 2, F // 2)),
            full((1, F // 2)),
            full((1, F // 2)),
            full((F // 2, 2)),
            full((1, 2)),
        ],
        out_specs=pl.BlockSpec((1, B), lambda i: (0, 0)),
        out_shape=jax.ShapeDtypeStruct((1, B), jnp.float32),
        scratch_shapes=[pltpu.VMEM((16, B), jnp.float32)],
    )(s, v2, pos, nt2, seg2, am_pad,
      Wv, Ws, Wn, b1_0.reshape(1, F), W2_0, b2_0.reshape(1, 2 * (F // 2)),
      Wc, W11a, W11b, b1_1.reshape(1, F // 2), W2_1, b2_1.reshape(1, 2))
    return out.reshape(B, 1)
